# baseline (speedup 1.0000x reference)
import functools

import jax
import jax.numpy as jnp
from jax import lax
from jax.experimental import pallas as pl
from jax.experimental.pallas import tpu as pltpu
from jax.experimental.pallas import tpu_sc as plsc

N_NODE = 10000
D = 128
H = 8
DH = 16
PD = 16

E_PAD = 163840
NTILES = 32
EPT = E_PAD // NTILES
BLK = 128
NBLK = EPT // BLK
ROWS_PER_TILE = N_NODE // 16


def _node_kernel(x_ref, g_ref, b_ref, wq_ref, wk_ref, wv_ref, m_ref,
                 qp_ref, kp_ref, vw_ref):
    x = x_ref[...]
    m = jnp.mean(x, axis=-1, keepdims=True)
    v = jnp.mean(x * x, axis=-1, keepdims=True) - m * m
    xn = (x - m) * lax.rsqrt(v + 1e-5) * g_ref[...] + b_ref[...]
    qp_ref[...] = jnp.dot(xn, wq_ref[...], preferred_element_type=jnp.float32)
    kp_ref[...] = jnp.dot(xn, wk_ref[...], preferred_element_type=jnp.float32)
    vp = jnp.dot(xn, wv_ref[...], preferred_element_type=jnp.float32)
    vw = jnp.dot(vp, m_ref[...], preferred_element_type=jnp.float32)
    col = lax.broadcasted_iota(jnp.int32, vw.shape, 1)
    vw_ref[...] = jnp.where(col < 8, 1.0, vw)


def _node_precompute(query, ln_q_g, ln_q_b, wqt_s, wkt, wvt, m_mat):
    nblk = 5
    rb = N_NODE // nblk
    full = lambda i: (0, 0)
    return pl.pallas_call(
        _node_kernel,
        grid=(nblk,),
        in_specs=[
            pl.BlockSpec((rb, D), lambda i: (i, 0)),
            pl.BlockSpec((1, D), full),
            pl.BlockSpec((1, D), full),
            pl.BlockSpec((D, D), full),
            pl.BlockSpec((D, D), full),
            pl.BlockSpec((D, D), full),
            pl.BlockSpec((D, 32), full),
        ],
        out_specs=[
            pl.BlockSpec((rb, D), lambda i: (i, 0)),
            pl.BlockSpec((rb, D), lambda i: (i, 0)),
            pl.BlockSpec((rb, 32), lambda i: (i, 0)),
        ],
        out_shape=[
            jax.ShapeDtypeStruct((N_NODE, D), jnp.float32),
            jax.ShapeDtypeStruct((N_NODE, D), jnp.float32),
            jax.ShapeDtypeStruct((N_NODE, 32), jnp.float32),
        ],
    )(query, ln_q_g, ln_q_b, wqt_s, wkt, wvt, m_mat)


def _edge_kernel(x0_ref, x1_ref, x2_ref, x3_ref,
                 d0_ref, d1_ref, d2_ref, d3_ref,
                 g4_ref, b4_ref, gm_ref, w4_ref, blb4_ref,
                 a_ref, bm_ref, mx_ref):
    xp = jnp.concatenate(
        [x0_ref[...], x1_ref[...], x2_ref[...], x3_ref[...]], axis=1)
    df = jnp.concatenate(
        [d0_ref[...], d1_ref[...], d2_ref[...], d3_ref[...]], axis=1)
    gm = gm_ref[...]
    mu = jnp.dot(xp, gm, preferred_element_type=jnp.float32)
    m2 = jnp.dot(xp * xp, gm, preferred_element_type=jnp.float32)
    var = m2 - mu * mu
    pn = (xp - mu) * lax.rsqrt(var + 1e-5) * g4_ref[...] + b4_ref[...]
    eb = jnp.exp(jnp.dot(pn, w4_ref[...], preferred_element_type=jnp.float32)
                 + blb4_ref[...])
    col = lax.broadcasted_iota(jnp.int32, (1, 128), 1)
    ones_lanes = jnp.where(col % 32 < 8, 1.0, 0.0)
    mx = ((jnp.dot(df, a_ref[...], preferred_element_type=jnp.float32)
           + ones_lanes)
          * jnp.dot(eb, bm_ref[...], preferred_element_type=jnp.float32))
    pad = mx_ref.shape[0] - mx.shape[0]
    mx_ref[...] = jnp.concatenate(
        [mx, jnp.zeros((pad, 128), jnp.float32)], axis=0)


def _edge_precompute(pair, ediff, g4, b4, gm, w4, blb4, a_mat, b_mat):
    nblk = 10
    rows_in = pair.shape[0] // (4 * nblk)
    rows_out = (E_PAD // 4) // nblk
    full = lambda i: (0, 0)
    pair_specs = [
        pl.BlockSpec((rows_in, PD), lambda i, c=c: (c * nblk + i, 0))
        for c in range(4)]
    dif_specs = [
        pl.BlockSpec((rows_in, 3), lambda i, c=c: (c * nblk + i, 0))
        for c in range(4)]
    return pl.pallas_call(
        _edge_kernel,
        grid=(nblk,),
        in_specs=pair_specs + dif_specs + [
            pl.BlockSpec((1, 64), full),
            pl.BlockSpec((1, 64), full),
            pl.BlockSpec((64, 64), full),
            pl.BlockSpec((64, 32), full),
            pl.BlockSpec((1, 32), full),
            pl.BlockSpec((12, 128), full),
            pl.BlockSpec((32, 128), full),
        ],
        out_specs=pl.BlockSpec((rows_out, 128), lambda i: (i, 0)),
        out_shape=jax.ShapeDtypeStruct((E_PAD // 4, 128), jnp.float32),
    )(pair, pair, pair, pair, ediff, ediff, ediff, ediff,
      g4, b4, gm, w4, blb4, a_mat, b_mat)


def _sc_edge_body(qp_hbm, kp_hbm, vw_hbm, mx_hbm, src_hbm, dst_hbm, part_hbm,
                  acc_sh,
                  srcb0, dstb0, qb0, kb0, vwb0, mxb0,
                  srcb1, dstb1, qb1, kb1, vwb1, mxb1,
                  cb, sem_a, sem_b):
    c = lax.axis_index("c")
    s = lax.axis_index("s")
    tid = c * 16 + s
    base = tid * EPT

    def zrow(r, carry):
        cb[r, pl.ds(0, 16)] = jnp.zeros((16,), jnp.float32)
        cb[r, pl.ds(16, 16)] = jnp.zeros((16,), jnp.float32)
        return carry

    lax.fori_loop(0, BLK, zrow, 0)
    nbase = s * ROWS_PER_TILE
    for t in range(4):
        pltpu.sync_copy(cb, acc_sh.at[pl.ds(nbase + t * BLK, BLK), :])
    pltpu.sync_copy(cb.at[pl.ds(0, 113), :],
                    acc_sh.at[pl.ds(nbase + 512, 113), :])
    plsc.subcore_barrier()

    lanes = lax.iota(jnp.int32, 16)
    dn = lax.GatherDimensionNumbers(offset_dims=(), collapsed_slice_dims=(0,),
                                    start_index_map=(0,))

    def tk(v, perm):
        return lax.gather(v, perm[:, None], dn, (1,),
                          mode=lax.GatherScatterMode.PROMISE_IN_BOUNDS)

    rot8 = (lanes + 8) & 15
    rot4h = (lanes & 8) | ((lanes + 4) & 7)
    rot2q = (lanes & 12) | ((lanes + 2) & 3)
    swap1 = lanes ^ 1
    l7 = lanes & 7
    permf = (((l7 & 1) << 2) | (l7 & 2) | ((l7 & 4) >> 2)) * 2
    m8 = lanes < 8
    m4 = (lanes & 7) < 4
    m2 = (lanes & 3) < 2

    def merge(x, y, mask_lt, rot):
        return (jnp.where(mask_lt, x, tk(y, rot))
                + jnp.where(mask_lt, tk(x, rot), y))

    def make_edge_body(qb, kb, vwb, mxb):
        def edge_body(e, carry):
            p = [qb[e, pl.ds(h * DH, DH)] * kb[e, pl.ds(h * DH, DH)]
                 for h in range(H)]
            r = [merge(p[2 * i], p[2 * i + 1], m8, rot8) for i in range(4)]
            sL = [merge(r[2 * j], r[2 * j + 1], m4, rot4h) for j in range(2)]
            t = merge(sL[0], sL[1], m2, rot2q)
            u = t + tk(t, swap1)
            sv = tk(u, permf)
            exd = jnp.exp(sv)
            r4 = e >> 2
            c4 = (e & 3) * 32
            cb[e, pl.ds(0, 16)] = exd * vwb[e, pl.ds(0, 16)] * mxb[r4, pl.ds(c4, 16)]
            cb[e, pl.ds(16, 16)] = exd * vwb[e, pl.ds(16, 16)] * mxb[r4, pl.ds(c4 + 16, 16)]
            return carry
        return edge_body

    eb0 = make_edge_body(qb0, kb0, vwb0, mxb0)
    eb1 = make_edge_body(qb1, kb1, vwb1, mxb1)

    def fire(j, srcb, dstb, qb, kb, vwb, mxb, sem):
        off = base + j * BLK
        pltpu.sync_copy(src_hbm.at[pl.ds(off, BLK)], srcb)
        pltpu.sync_copy(dst_hbm.at[pl.ds(off, BLK)], dstb)
        pltpu.async_copy(qp_hbm.at[srcb], qb, sem)
        pltpu.async_copy(kp_hbm.at[dstb], kb, sem)
        pltpu.async_copy(vw_hbm.at[dstb], vwb, sem)
        pltpu.sync_copy(mx_hbm.at[pl.ds(off // 4, BLK // 4), :], mxb)

    def drain(qb, kb, vwb, sem):
        pltpu.make_async_copy(qp_hbm.at[pl.ds(0, BLK), :], qb, sem).wait()
        pltpu.make_async_copy(kp_hbm.at[pl.ds(0, BLK), :], kb, sem).wait()
        pltpu.make_async_copy(vw_hbm.at[pl.ds(0, BLK), :], vwb, sem).wait()

    fire(0, srcb0, dstb0, qb0, kb0, vwb0, mxb0, sem_a)

    def pair_body(pr, carry):
        j0 = 2 * pr
        fire(j0 + 1, srcb1, dstb1, qb1, kb1, vwb1, mxb1, sem_b)
        drain(qb0, kb0, vwb0, sem_a)
        lax.fori_loop(0, BLK, eb0, 0, unroll=2)
        pltpu.sync_copy(cb, acc_sh.at[srcb0], add=True)

        @pl.when(j0 + 2 < NBLK)
        def _():
            fire(j0 + 2, srcb0, dstb0, qb0, kb0, vwb0, mxb0, sem_a)

        drain(qb1, kb1, vwb1, sem_b)
        lax.fori_loop(0, BLK, eb1, 0, unroll=2)
        pltpu.sync_copy(cb, acc_sh.at[srcb1], add=True)
        return carry

    lax.fori_loop(0, NBLK // 2, pair_body, 0)

    plsc.subcore_barrier()
    rows = pl.ds(nbase, ROWS_PER_TILE)
    pltpu.sync_copy(acc_sh.at[rows, :], part_hbm.at[c, rows, :])


def _sc_edge_phase(qp, kp, vw, mx, src, dst):
    mesh = plsc.VectorSubcoreMesh(core_axis_name="c", subcore_axis_name="s")
    f = pl.kernel(
        _sc_edge_body,
        mesh=mesh,
        compiler_params=pltpu.CompilerParams(use_tc_tiling_on_sc=False),
        out_type=jax.ShapeDtypeStruct((2, N_NODE, 32), jnp.float32),
        scratch_types=[
            pltpu.VMEM_SHARED((N_NODE, 32), jnp.float32),
            pltpu.VMEM((BLK,), jnp.int32),
            pltpu.VMEM((BLK,), jnp.int32),
            pltpu.VMEM((BLK, D), jnp.float32),
            pltpu.VMEM((BLK, D), jnp.float32),
            pltpu.VMEM((BLK, 32), jnp.float32),
            pltpu.VMEM((BLK // 4, 128), jnp.float32),
            pltpu.VMEM((BLK,), jnp.int32),
            pltpu.VMEM((BLK,), jnp.int32),
            pltpu.VMEM((BLK, D), jnp.float32),
            pltpu.VMEM((BLK, D), jnp.float32),
            pltpu.VMEM((BLK, 32), jnp.float32),
            pltpu.VMEM((BLK // 4, 128), jnp.float32),
            pltpu.VMEM((BLK, 32), jnp.float32),
            pltpu.SemaphoreType.DMA,
            pltpu.SemaphoreType.DMA,
        ],
    )
    return f(qp, kp, vw, mx, src, dst)


def _combine_kernel(p_ref, bf_ref, o_ref):
    x = p_ref[0] + p_ref[1]
    denom = x[:, 0:8]
    safe = jnp.where(denom != 0.0, denom, 1.0)
    safe3 = jnp.concatenate([safe, safe, safe], axis=1)
    y = x[:, 8:32] / safe3
    r = lax.broadcasted_iota(jnp.int32, (24, 3), 0) // 8
    col = lax.broadcasted_iota(jnp.int32, (24, 3), 1)
    ssum = jnp.where(r == col, 1.0, 0.0)
    o_ref[...] = jnp.dot(y, ssum, preferred_element_type=jnp.float32) + bf_ref[...]


def _combine(partial, bfv):
    nblk = 10
    rb = N_NODE // nblk
    return pl.pallas_call(
        _combine_kernel,
        grid=(nblk,),
        in_specs=[
            pl.BlockSpec((2, rb, 32), lambda i: (0, i, 0)),
            pl.BlockSpec((1, 3), lambda i: (0, 0)),
        ],
        out_specs=pl.BlockSpec((rb, 3), lambda i: (i, 0)),
        out_shape=jax.ShapeDtypeStruct((N_NODE, 3), jnp.float32),
    )(partial, bfv)


def kernel(query, edge_index, edge_diff, pair, ln_q_g, ln_q_b, ln_p_g, ln_p_b,
           Wq, Wk, Wv, Wlb, blb, Wf1, bf1, Wf2, bf2, Wf3, bf3):
    scaling = DH ** -0.5

    wqt_s = Wq.T * scaling
    wkt = Wk.T
    wvt = Wv.T
    dd = jnp.arange(D)
    hh = jnp.arange(H)
    head_mask = jnp.where(dd[:, None] // DH == hh[None, :], 1.0, 0.0)
    m_mat = jnp.concatenate(
        [jnp.zeros((D, 8), jnp.float32),
         Wf1[0][:, None] * head_mask,
         Wf2[0][:, None] * head_mask,
         Wf3[0][:, None] * head_mask], axis=1)

    rr64 = jnp.arange(64)
    gm = jnp.where(rr64[:, None] // 16 == rr64[None, :] // 16, 1.0 / 16, 0.0)
    cc32 = jnp.arange(32)
    w4 = (jnp.tile(Wlb.T, (4, 4))
          * jnp.where(rr64[:, None] // 16 == cc32[None, :] // 8, 1.0, 0.0))
    blb4 = jnp.tile(blb, 4).reshape(1, 32)
    g4 = jnp.tile(ln_p_g, 4).reshape(1, 64)
    b4 = jnp.tile(ln_p_b, 4).reshape(1, 64)
    rr12 = jnp.arange(12)
    cc128 = jnp.arange(128)
    a_mat = jnp.where((cc128[None, :] // 32 == rr12[:, None] // 3)
                      & ((cc128[None, :] % 32) // 8 == rr12[:, None] % 3 + 1),
                      1.0, 0.0)
    rr32 = jnp.arange(32)
    b_mat = jnp.where((cc128[None, :] // 32 == rr32[:, None] // 8)
                      & (cc128[None, :] % 8 == rr32[:, None] % 8),
                      1.0, 0.0)

    n_edge = edge_index.shape[0]
    npad = E_PAD - n_edge
    nblk2 = 10
    rows_in = n_edge // (4 * nblk2)
    rows_out = (E_PAD // 4) // nblk2
    pad_rows = rows_out - rows_in
    pad_ids = (jnp.arange(4 * nblk2 * pad_rows, dtype=jnp.int32)
               % N_NODE).reshape(4, nblk2, pad_rows)
    ei = edge_index.astype(jnp.int32)

    def permute(col):
        full = jnp.concatenate(
            [col.reshape(4, nblk2, rows_in), pad_ids], axis=2)
        return jnp.transpose(full, (1, 2, 0)).reshape(E_PAD)

    src = permute(ei[:, 0])
    dst = permute(ei[:, 1])
    bfv = jnp.concatenate([bf1, bf2, bf3]).reshape(1, 3)

    qp, kp, vw = _node_precompute(query, ln_q_g.reshape(1, D),
                                  ln_q_b.reshape(1, D), wqt_s, wkt, wvt, m_mat)
    mx = _edge_precompute(pair, edge_diff, g4, b4, gm, w4, blb4,
                          a_mat, b_mat)
    partial = _sc_edge_phase(qp, kp, vw, mx, src, dst)
    return _combine(partial, bfv)


# speedup vs baseline: 1.1683x; 1.1683x over previous
import functools

import jax
import jax.numpy as jnp
from jax import lax
from jax.experimental import pallas as pl
from jax.experimental.pallas import tpu as pltpu
from jax.experimental.pallas import tpu_sc as plsc

N_NODE = 10000
D = 128
H = 8
DH = 16
PD = 16

E_PAD = 163840
NTILES = 32
EPT = E_PAD // NTILES
BLK = 128
NBLK = EPT // BLK
ROWS_PER_TILE = N_NODE // 16


def _node_kernel(x_ref, g_ref, b_ref, wq_ref, wk_ref, wv_ref, m_ref,
                 qp_ref, kp_ref, vw_ref):
    x = x_ref[...]
    m = jnp.mean(x, axis=-1, keepdims=True)
    v = jnp.mean(x * x, axis=-1, keepdims=True) - m * m
    xn = (x - m) * lax.rsqrt(v + 1e-5) * g_ref[...] + b_ref[...]
    qp_ref[...] = jnp.dot(xn, wq_ref[...], preferred_element_type=jnp.float32)
    kp_ref[...] = jnp.dot(xn, wk_ref[...], preferred_element_type=jnp.float32)
    vp = jnp.dot(xn, wv_ref[...], preferred_element_type=jnp.float32)
    vw = jnp.dot(vp, m_ref[...], preferred_element_type=jnp.float32)
    col = lax.broadcasted_iota(jnp.int32, vw.shape, 1)
    vw_ref[...] = jnp.where(col < 8, 1.0, vw)


def _node_precompute(query, ln_q_g, ln_q_b, wqt_s, wkt, wvt, m_mat):
    nblk = 5
    rb = N_NODE // nblk
    full = lambda i: (0, 0)
    return pl.pallas_call(
        _node_kernel,
        grid=(nblk,),
        in_specs=[
            pl.BlockSpec((rb, D), lambda i: (i, 0)),
            pl.BlockSpec((1, D), full),
            pl.BlockSpec((1, D), full),
            pl.BlockSpec((D, D), full),
            pl.BlockSpec((D, D), full),
            pl.BlockSpec((D, D), full),
            pl.BlockSpec((D, 32), full),
        ],
        out_specs=[
            pl.BlockSpec((rb, D), lambda i: (i, 0)),
            pl.BlockSpec((rb, D), lambda i: (i, 0)),
            pl.BlockSpec((rb, 32), lambda i: (i, 0)),
        ],
        out_shape=[
            jax.ShapeDtypeStruct((N_NODE, D), jnp.float32),
            jax.ShapeDtypeStruct((N_NODE, D), jnp.float32),
            jax.ShapeDtypeStruct((N_NODE, 32), jnp.float32),
        ],
    )(query, ln_q_g, ln_q_b, wqt_s, wkt, wvt, m_mat)


def _edge_kernel(x0_ref, x1_ref, x2_ref, x3_ref,
                 d0_ref, d1_ref, d2_ref, d3_ref,
                 g4_ref, b4_ref, gm_ref, w4_ref, blb4_ref,
                 a_ref, bm_ref, mx_ref):
    row16 = lax.broadcasted_iota(jnp.int32, (16, 64), 0)
    col64 = lax.broadcasted_iota(jnp.int32, (16, 64), 1)
    xs = [x0_ref[...], x1_ref[...], x2_ref[...], x3_ref[...]]
    xp = sum(
        jnp.dot(xs[c], jnp.where(col64 == c * 16 + row16, 1.0, 0.0),
                preferred_element_type=jnp.float32)
        for c in range(4))
    ds_ = [d0_ref[...], d1_ref[...], d2_ref[...], d3_ref[...]]
    dfa = sum(
        jnp.dot(ds_[c], a_ref[c * 3:(c + 1) * 3, :],
                preferred_element_type=jnp.float32)
        for c in range(4))
    gm = gm_ref[...]
    mu = jnp.dot(xp, gm, preferred_element_type=jnp.float32)
    m2 = jnp.dot(xp * xp, gm, preferred_element_type=jnp.float32)
    var = m2 - mu * mu
    pn = (xp - mu) * lax.rsqrt(var + 1e-5) * g4_ref[...] + b4_ref[...]
    eb = jnp.exp(jnp.dot(pn, w4_ref[...], preferred_element_type=jnp.float32)
                 + blb4_ref[...])
    col = lax.broadcasted_iota(jnp.int32, (1, 128), 1)
    ones_lanes = jnp.where(col % 32 < 8, 1.0, 0.0)
    mx = ((dfa + ones_lanes)
          * jnp.dot(eb, bm_ref[...], preferred_element_type=jnp.float32))
    pad = mx_ref.shape[0] - mx.shape[0]
    mx_ref[...] = jnp.concatenate(
        [mx, jnp.zeros((pad, 128), jnp.float32)], axis=0)


def _edge_precompute(pair, ediff, g4, b4, gm, w4, blb4, a_mat, b_mat):
    nblk = 10
    rows_in = pair.shape[0] // (4 * nblk)
    rows_out = (E_PAD // 4) // nblk
    full = lambda i: (0, 0)
    pair_specs = [
        pl.BlockSpec((rows_in, PD), lambda i, c=c: (c * nblk + i, 0))
        for c in range(4)]
    dif_specs = [
        pl.BlockSpec((rows_in, 3), lambda i, c=c: (c * nblk + i, 0))
        for c in range(4)]
    return pl.pallas_call(
        _edge_kernel,
        grid=(nblk,),
        in_specs=pair_specs + dif_specs + [
            pl.BlockSpec((1, 64), full),
            pl.BlockSpec((1, 64), full),
            pl.BlockSpec((64, 64), full),
            pl.BlockSpec((64, 32), full),
            pl.BlockSpec((1, 32), full),
            pl.BlockSpec((12, 128), full),
            pl.BlockSpec((32, 128), full),
        ],
        out_specs=pl.BlockSpec((rows_out, 128), lambda i: (i, 0)),
        out_shape=jax.ShapeDtypeStruct((E_PAD // 4, 128), jnp.float32),
    )(pair, pair, pair, pair, ediff, ediff, ediff, ediff,
      g4, b4, gm, w4, blb4, a_mat, b_mat)


def _sc_edge_body(qp_hbm, kp_hbm, vw_hbm, mx_hbm, src_hbm, dst_hbm, part_hbm,
                  acc_sh,
                  srcb0, dstb0, qb0, kb0, vwb0, mxb0,
                  srcb1, dstb1, qb1, kb1, vwb1, mxb1,
                  cb, sem_a, sem_b):
    c = lax.axis_index("c")
    s = lax.axis_index("s")
    tid = c * 16 + s
    base = tid * EPT
    cq = tid // 8

    def zrow(r, carry):
        cb[r, pl.ds(0, 16)] = jnp.zeros((16,), jnp.float32)
        cb[r, pl.ds(16, 16)] = jnp.zeros((16,), jnp.float32)
        return carry

    lax.fori_loop(0, BLK, zrow, 0)
    nbase = s * ROWS_PER_TILE
    for t in range(4):
        pltpu.sync_copy(cb, acc_sh.at[pl.ds(nbase + t * BLK, BLK), :])
    pltpu.sync_copy(cb.at[pl.ds(0, 113), :],
                    acc_sh.at[pl.ds(nbase + 512, 113), :])
    plsc.subcore_barrier()

    lanes = lax.iota(jnp.int32, 16)
    dn = lax.GatherDimensionNumbers(offset_dims=(), collapsed_slice_dims=(0,),
                                    start_index_map=(0,))

    def tk(v, perm):
        return lax.gather(v, perm[:, None], dn, (1,),
                          mode=lax.GatherScatterMode.PROMISE_IN_BOUNDS)

    rot8 = (lanes + 8) & 15
    rot4h = (lanes & 8) | ((lanes + 4) & 7)
    rot2q = (lanes & 12) | ((lanes + 2) & 3)
    swap1 = lanes ^ 1
    l7 = lanes & 7
    permf = (((l7 & 1) << 2) | (l7 & 2) | ((l7 & 4) >> 2)) * 2
    m8 = lanes < 8
    m4 = (lanes & 7) < 4
    m2 = (lanes & 3) < 2

    def merge(x, y, mask_lt, rot):
        return (jnp.where(mask_lt, x, tk(y, rot))
                + jnp.where(mask_lt, tk(x, rot), y))

    def make_edge_body(qb, kb, vwb, mxb):
        def edge_body(e, carry):
            p = [qb[e, pl.ds(h * DH, DH)] * kb[e, pl.ds(h * DH, DH)]
                 for h in range(H)]
            r = [merge(p[2 * i], p[2 * i + 1], m8, rot8) for i in range(4)]
            sL = [merge(r[2 * j], r[2 * j + 1], m4, rot4h) for j in range(2)]
            t = merge(sL[0], sL[1], m2, rot2q)
            u = t + tk(t, swap1)
            sv = tk(u, permf)
            exd = jnp.exp(sv)
            cb[e, pl.ds(0, 16)] = exd * vwb[e, pl.ds(0, 16)] * mxb[e, pl.ds(0, 16)]
            cb[e, pl.ds(16, 16)] = exd * vwb[e, pl.ds(16, 16)] * mxb[e, pl.ds(16, 16)]
            return carry
        return edge_body

    eb0 = make_edge_body(qb0, kb0, vwb0, mxb0)
    eb1 = make_edge_body(qb1, kb1, vwb1, mxb1)

    def fire(j, srcb, dstb, qb, kb, vwb, mxb, sem):
        off = base + j * BLK
        pltpu.sync_copy(src_hbm.at[pl.ds(off, BLK)], srcb)
        pltpu.sync_copy(dst_hbm.at[pl.ds(off, BLK)], dstb)
        pltpu.async_copy(qp_hbm.at[srcb], qb, sem)
        pltpu.async_copy(kp_hbm.at[dstb], kb, sem)
        pltpu.async_copy(vw_hbm.at[dstb], vwb, sem)
        pltpu.sync_copy(
            mx_hbm.at[pl.ds(off - cq * (8 * EPT), BLK), pl.ds(cq * 32, 32)],
            mxb)

    def drain(qb, kb, vwb, sem):
        pltpu.make_async_copy(qp_hbm.at[pl.ds(0, BLK), :], qb, sem).wait()
        pltpu.make_async_copy(kp_hbm.at[pl.ds(0, BLK), :], kb, sem).wait()
        pltpu.make_async_copy(vw_hbm.at[pl.ds(0, BLK), :], vwb, sem).wait()

    fire(0, srcb0, dstb0, qb0, kb0, vwb0, mxb0, sem_a)

    def pair_body(pr, carry):
        j0 = 2 * pr
        fire(j0 + 1, srcb1, dstb1, qb1, kb1, vwb1, mxb1, sem_b)
        drain(qb0, kb0, vwb0, sem_a)
        lax.fori_loop(0, BLK, eb0, 0)
        pltpu.sync_copy(cb, acc_sh.at[srcb0], add=True)

        @pl.when(j0 + 2 < NBLK)
        def _():
            fire(j0 + 2, srcb0, dstb0, qb0, kb0, vwb0, mxb0, sem_a)

        drain(qb1, kb1, vwb1, sem_b)
        lax.fori_loop(0, BLK, eb1, 0)
        pltpu.sync_copy(cb, acc_sh.at[srcb1], add=True)
        return carry

    lax.fori_loop(0, NBLK // 2, pair_body, 0)

    plsc.subcore_barrier()
    rows = pl.ds(nbase, ROWS_PER_TILE)
    pltpu.sync_copy(acc_sh.at[rows, :], part_hbm.at[c, rows, :])


def _sc_edge_phase(qp, kp, vw, mx, src, dst):
    mesh = plsc.VectorSubcoreMesh(core_axis_name="c", subcore_axis_name="s")
    f = pl.kernel(
        _sc_edge_body,
        mesh=mesh,
        compiler_params=pltpu.CompilerParams(use_tc_tiling_on_sc=False),
        out_type=jax.ShapeDtypeStruct((2, N_NODE, 32), jnp.float32),
        scratch_types=[
            pltpu.VMEM_SHARED((N_NODE, 32), jnp.float32),
            pltpu.VMEM((BLK,), jnp.int32),
            pltpu.VMEM((BLK,), jnp.int32),
            pltpu.VMEM((BLK, D), jnp.float32),
            pltpu.VMEM((BLK, D), jnp.float32),
            pltpu.VMEM((BLK, 32), jnp.float32),
            pltpu.VMEM((BLK, 32), jnp.float32),
            pltpu.VMEM((BLK,), jnp.int32),
            pltpu.VMEM((BLK,), jnp.int32),
            pltpu.VMEM((BLK, D), jnp.float32),
            pltpu.VMEM((BLK, D), jnp.float32),
            pltpu.VMEM((BLK, 32), jnp.float32),
            pltpu.VMEM((BLK, 32), jnp.float32),
            pltpu.VMEM((BLK, 32), jnp.float32),
            pltpu.SemaphoreType.DMA,
            pltpu.SemaphoreType.DMA,
        ],
    )
    return f(qp, kp, vw, mx, src, dst)


def _combine_kernel(p_ref, bf_ref, o_ref):
    x = p_ref[0] + p_ref[1]
    denom = x[:, 0:8]
    safe = jnp.where(denom != 0.0, denom, 1.0)
    safe3 = jnp.concatenate([safe, safe, safe], axis=1)
    y = x[:, 8:32] / safe3
    r = lax.broadcasted_iota(jnp.int32, (24, 3), 0) // 8
    col = lax.broadcasted_iota(jnp.int32, (24, 3), 1)
    ssum = jnp.where(r == col, 1.0, 0.0)
    o_ref[...] = jnp.dot(y, ssum, preferred_element_type=jnp.float32) + bf_ref[...]


def _combine(partial, bfv):
    nblk = 10
    rb = N_NODE // nblk
    return pl.pallas_call(
        _combine_kernel,
        grid=(nblk,),
        in_specs=[
            pl.BlockSpec((2, rb, 32), lambda i: (0, i, 0)),
            pl.BlockSpec((1, 3), lambda i: (0, 0)),
        ],
        out_specs=pl.BlockSpec((rb, 3), lambda i: (i, 0)),
        out_shape=jax.ShapeDtypeStruct((N_NODE, 3), jnp.float32),
    )(partial, bfv)


def kernel(query, edge_index, edge_diff, pair, ln_q_g, ln_q_b, ln_p_g, ln_p_b,
           Wq, Wk, Wv, Wlb, blb, Wf1, bf1, Wf2, bf2, Wf3, bf3):
    scaling = DH ** -0.5

    wqt_s = Wq.T * scaling
    wkt = Wk.T
    wvt = Wv.T
    dd = jnp.arange(D)
    hh = jnp.arange(H)
    head_mask = jnp.where(dd[:, None] // DH == hh[None, :], 1.0, 0.0)
    m_mat = jnp.concatenate(
        [jnp.zeros((D, 8), jnp.float32),
         Wf1[0][:, None] * head_mask,
         Wf2[0][:, None] * head_mask,
         Wf3[0][:, None] * head_mask], axis=1)

    rr64 = jnp.arange(64)
    gm = jnp.where(rr64[:, None] // 16 == rr64[None, :] // 16, 1.0 / 16, 0.0)
    cc32 = jnp.arange(32)
    w4 = (jnp.tile(Wlb.T, (4, 4))
          * jnp.where(rr64[:, None] // 16 == cc32[None, :] // 8, 1.0, 0.0))
    blb4 = jnp.tile(blb, 4).reshape(1, 32)
    g4 = jnp.tile(ln_p_g, 4).reshape(1, 64)
    b4 = jnp.tile(ln_p_b, 4).reshape(1, 64)
    rr12 = jnp.arange(12)
    cc128 = jnp.arange(128)
    a_mat = jnp.where((cc128[None, :] // 32 == rr12[:, None] // 3)
                      & ((cc128[None, :] % 32) // 8 == rr12[:, None] % 3 + 1),
                      1.0, 0.0)
    rr32 = jnp.arange(32)
    b_mat = jnp.where((cc128[None, :] // 32 == rr32[:, None] // 8)
                      & (cc128[None, :] % 8 == rr32[:, None] % 8),
                      1.0, 0.0)

    n_edge = edge_index.shape[0]
    npad = E_PAD - n_edge
    nblk2 = 10
    rows_in = n_edge // (4 * nblk2)
    rows_out = (E_PAD // 4) // nblk2
    pad_rows = rows_out - rows_in
    pad_ids = (jnp.arange(4 * nblk2 * pad_rows, dtype=jnp.int32)
               % N_NODE).reshape(4, nblk2, pad_rows)
    ei = edge_index.astype(jnp.int32)

    def permute(col):
        full = jnp.concatenate(
            [col.reshape(4, nblk2, rows_in), pad_ids], axis=2)
        return full.reshape(E_PAD)

    src = permute(ei[:, 0])
    dst = permute(ei[:, 1])
    bfv = jnp.concatenate([bf1, bf2, bf3]).reshape(1, 3)

    qp, kp, vw = _node_precompute(query, ln_q_g.reshape(1, D),
                                  ln_q_b.reshape(1, D), wqt_s, wkt, wvt, m_mat)
    mx = _edge_precompute(pair, edge_diff, g4, b4, gm, w4, blb4,
                          a_mat, b_mat)
    partial = _sc_edge_phase(qp, kp, vw, mx, src, dst)
    return _combine(partial, bfv)


# speedup vs baseline: 1.5381x; 1.3165x over previous
import functools

import jax
import jax.numpy as jnp
from jax import lax
from jax.experimental import pallas as pl
from jax.experimental.pallas import tpu as pltpu
from jax.experimental.pallas import tpu_sc as plsc

N_NODE = 10000
D = 128
H = 8
DH = 16
PD = 16

E_PAD = 163840
NTILES = 32
EPT = E_PAD // NTILES
BLK = 128
NBLK = EPT // BLK
ROWS_PER_TILE = N_NODE // 16


def _node_kernel(x_ref, g_ref, b_ref, wq_ref, wk_ref, wv_ref, m_ref,
                 qp_ref, kp_ref, vw_ref):
    x = x_ref[...]
    m = jnp.mean(x, axis=-1, keepdims=True)
    v = jnp.mean(x * x, axis=-1, keepdims=True) - m * m
    xn = (x - m) * lax.rsqrt(v + 1e-5) * g_ref[...] + b_ref[...]
    qp_ref[...] = jnp.dot(xn, wq_ref[...], preferred_element_type=jnp.float32)
    kp_ref[...] = jnp.dot(xn, wk_ref[...], preferred_element_type=jnp.float32)
    vp = jnp.dot(xn, wv_ref[...], preferred_element_type=jnp.float32)
    vw = jnp.dot(vp, m_ref[...], preferred_element_type=jnp.float32)
    col = lax.broadcasted_iota(jnp.int32, vw.shape, 1)
    vw_ref[...] = jnp.where(col < 8, 1.0, vw)


def _node_precompute(query, ln_q_g, ln_q_b, wqt_s, wkt, wvt, m_mat):
    nblk = 5
    rb = N_NODE // nblk
    full = lambda i: (0, 0)
    return pl.pallas_call(
        _node_kernel,
        grid=(nblk,),
        in_specs=[
            pl.BlockSpec((rb, D), lambda i: (i, 0)),
            pl.BlockSpec((1, D), full),
            pl.BlockSpec((1, D), full),
            pl.BlockSpec((D, D), full),
            pl.BlockSpec((D, D), full),
            pl.BlockSpec((D, D), full),
            pl.BlockSpec((D, 32), full),
        ],
        out_specs=[
            pl.BlockSpec((rb, D), lambda i: (i, 0)),
            pl.BlockSpec((rb, D), lambda i: (i, 0)),
            pl.BlockSpec((rb, 32), lambda i: (i, 0)),
        ],
        out_shape=[
            jax.ShapeDtypeStruct((N_NODE, D), jnp.float32),
            jax.ShapeDtypeStruct((N_NODE, D), jnp.float32),
            jax.ShapeDtypeStruct((N_NODE, 32), jnp.float32),
        ],
    )(query, ln_q_g, ln_q_b, wqt_s, wkt, wvt, m_mat)


def _edge_kernel(x0_ref, x1_ref, x2_ref, x3_ref,
                 d0_ref, d1_ref, d2_ref, d3_ref,
                 g4_ref, b4_ref, gm_ref, w4_ref, blb4_ref,
                 a_ref, bm_ref, mx_ref):
    row16 = lax.broadcasted_iota(jnp.int32, (16, 64), 0)
    col64 = lax.broadcasted_iota(jnp.int32, (16, 64), 1)
    xs = [x0_ref[...], x1_ref[...], x2_ref[...], x3_ref[...]]
    xp = sum(
        jnp.dot(xs[c], jnp.where(col64 == c * 16 + row16, 1.0, 0.0),
                preferred_element_type=jnp.float32)
        for c in range(4))
    ds_ = [d0_ref[...], d1_ref[...], d2_ref[...], d3_ref[...]]
    dfa = sum(
        jnp.dot(ds_[c], a_ref[c * 3:(c + 1) * 3, :],
                preferred_element_type=jnp.float32)
        for c in range(4))
    gm = gm_ref[...]
    mu = jnp.dot(xp, gm, preferred_element_type=jnp.float32)
    m2 = jnp.dot(xp * xp, gm, preferred_element_type=jnp.float32)
    var = m2 - mu * mu
    pn = (xp - mu) * lax.rsqrt(var + 1e-5) * g4_ref[...] + b4_ref[...]
    eb = jnp.exp(jnp.dot(pn, w4_ref[...], preferred_element_type=jnp.float32)
                 + blb4_ref[...])
    col = lax.broadcasted_iota(jnp.int32, (1, 128), 1)
    ones_lanes = jnp.where(col % 32 < 8, 1.0, 0.0)
    mx = ((dfa + ones_lanes)
          * jnp.dot(eb, bm_ref[...], preferred_element_type=jnp.float32))
    pad = mx_ref.shape[0] - mx.shape[0]
    mx_ref[...] = jnp.concatenate(
        [mx, jnp.zeros((pad, 128), jnp.float32)], axis=0)


def _edge_precompute(pair, ediff, g4, b4, gm, w4, blb4, a_mat, b_mat):
    nblk = 10
    rows_in = pair.shape[0] // (4 * nblk)
    rows_out = (E_PAD // 4) // nblk
    full = lambda i: (0, 0)
    pair_specs = [
        pl.BlockSpec((rows_in, PD), lambda i, c=c: (c * nblk + i, 0))
        for c in range(4)]
    dif_specs = [
        pl.BlockSpec((rows_in, 3), lambda i, c=c: (c * nblk + i, 0))
        for c in range(4)]
    return pl.pallas_call(
        _edge_kernel,
        grid=(nblk,),
        in_specs=pair_specs + dif_specs + [
            pl.BlockSpec((1, 64), full),
            pl.BlockSpec((1, 64), full),
            pl.BlockSpec((64, 64), full),
            pl.BlockSpec((64, 32), full),
            pl.BlockSpec((1, 32), full),
            pl.BlockSpec((12, 128), full),
            pl.BlockSpec((32, 128), full),
        ],
        out_specs=pl.BlockSpec((rows_out, 128), lambda i: (i, 0)),
        out_shape=jax.ShapeDtypeStruct((E_PAD // 4, 128), jnp.float32),
    )(pair, pair, pair, pair, ediff, ediff, ediff, ediff,
      g4, b4, gm, w4, blb4, a_mat, b_mat)


def _sc_edge_body(qp_hbm, kp_hbm, vw_hbm, mx_hbm, src_hbm, dst_hbm, part_hbm,
                  acc_sh,
                  srcb0, dstb0, qb0, kb0, vwb0, mxb0,
                  srcb1, dstb1, qb1, kb1, vwb1, mxb1,
                  cb, sem_a, sem_b):
    c = lax.axis_index("c")
    s = lax.axis_index("s")
    tid = c * 16 + s
    base = tid * EPT
    cq = tid // 8

    def zrow(r, carry):
        cb[r, pl.ds(0, 16)] = jnp.zeros((16,), jnp.float32)
        cb[r, pl.ds(16, 16)] = jnp.zeros((16,), jnp.float32)
        return carry

    lax.fori_loop(0, BLK, zrow, 0)
    nbase = s * ROWS_PER_TILE
    for t in range(4):
        pltpu.sync_copy(cb, acc_sh.at[pl.ds(nbase + t * BLK, BLK), :])
    pltpu.sync_copy(cb.at[pl.ds(0, 113), :],
                    acc_sh.at[pl.ds(nbase + 512, 113), :])
    plsc.subcore_barrier()

    lanes = lax.iota(jnp.int32, 16)
    dn = lax.GatherDimensionNumbers(offset_dims=(), collapsed_slice_dims=(0,),
                                    start_index_map=(0,))

    def tk(v, perm):
        return lax.gather(v, perm[:, None], dn, (1,),
                          mode=lax.GatherScatterMode.PROMISE_IN_BOUNDS)

    rot8 = (lanes + 8) & 15
    rot4h = (lanes & 8) | ((lanes + 4) & 7)
    rot2q = (lanes & 12) | ((lanes + 2) & 3)
    swap1 = lanes ^ 1
    l7 = lanes & 7
    permf = (((l7 & 1) << 2) | (l7 & 2) | ((l7 & 4) >> 2)) * 2
    m8 = lanes < 8
    m4 = (lanes & 7) < 4
    m2 = (lanes & 3) < 2

    def merge(x, y, mask_lt, rot):
        return (jnp.where(mask_lt, x, tk(y, rot))
                + jnp.where(mask_lt, tk(x, rot), y))

    def make_edge_body(qb, kb, vwb, mxb):
        def edge_body(e):
            p = [qb[e, pl.ds(h * DH, DH)] * kb[e, pl.ds(h * DH, DH)]
                 for h in range(H)]
            r = [merge(p[2 * i], p[2 * i + 1], m8, rot8) for i in range(4)]
            sL = [merge(r[2 * j], r[2 * j + 1], m4, rot4h) for j in range(2)]
            t = merge(sL[0], sL[1], m2, rot2q)
            u = t + tk(t, swap1)
            sv = tk(u, permf)
            exd = jnp.exp(sv)
            cb[e, pl.ds(0, 16)] = exd * vwb[e, pl.ds(0, 16)] * mxb[e, pl.ds(0, 16)]
            cb[e, pl.ds(16, 16)] = exd * vwb[e, pl.ds(16, 16)] * mxb[e, pl.ds(16, 16)]
        return edge_body

    eb0 = make_edge_body(qb0, kb0, vwb0, mxb0)
    eb1 = make_edge_body(qb1, kb1, vwb1, mxb1)

    def fire(j, srcb, dstb, qb, kb, vwb, mxb, sem):
        off = base + j * BLK
        pltpu.sync_copy(src_hbm.at[pl.ds(off, BLK)], srcb)
        pltpu.sync_copy(dst_hbm.at[pl.ds(off, BLK)], dstb)
        pltpu.async_copy(qp_hbm.at[srcb], qb, sem)
        pltpu.async_copy(kp_hbm.at[dstb], kb, sem)
        pltpu.async_copy(vw_hbm.at[dstb], vwb, sem)
        pltpu.sync_copy(
            mx_hbm.at[pl.ds(off - cq * (8 * EPT), BLK), pl.ds(cq * 32, 32)],
            mxb)

    def drain(qb, kb, vwb, sem):
        pltpu.make_async_copy(qp_hbm.at[pl.ds(0, BLK), :], qb, sem).wait()
        pltpu.make_async_copy(kp_hbm.at[pl.ds(0, BLK), :], kb, sem).wait()
        pltpu.make_async_copy(vw_hbm.at[pl.ds(0, BLK), :], vwb, sem).wait()

    fire(0, srcb0, dstb0, qb0, kb0, vwb0, mxb0, sem_a)

    def pair_body(pr, carry):
        j0 = 2 * pr
        fire(j0 + 1, srcb1, dstb1, qb1, kb1, vwb1, mxb1, sem_b)
        drain(qb0, kb0, vwb0, sem_a)
        plsc.parallel_loop(0, BLK, step=1, unroll=2)(eb0)
        pltpu.sync_copy(cb, acc_sh.at[srcb0], add=True)

        @pl.when(j0 + 2 < NBLK)
        def _():
            fire(j0 + 2, srcb0, dstb0, qb0, kb0, vwb0, mxb0, sem_a)

        drain(qb1, kb1, vwb1, sem_b)
        plsc.parallel_loop(0, BLK, step=1, unroll=2)(eb1)
        pltpu.sync_copy(cb, acc_sh.at[srcb1], add=True)
        return carry

    lax.fori_loop(0, NBLK // 2, pair_body, 0)

    plsc.subcore_barrier()
    rows = pl.ds(nbase, ROWS_PER_TILE)
    pltpu.sync_copy(acc_sh.at[rows, :], part_hbm.at[c, rows, :])


def _sc_edge_phase(qp, kp, vw, mx, src, dst):
    mesh = plsc.VectorSubcoreMesh(core_axis_name="c", subcore_axis_name="s")
    f = pl.kernel(
        _sc_edge_body,
        mesh=mesh,
        compiler_params=pltpu.CompilerParams(use_tc_tiling_on_sc=False),
        out_type=jax.ShapeDtypeStruct((2, N_NODE, 32), jnp.float32),
        scratch_types=[
            pltpu.VMEM_SHARED((N_NODE, 32), jnp.float32),
            pltpu.VMEM((BLK,), jnp.int32),
            pltpu.VMEM((BLK,), jnp.int32),
            pltpu.VMEM((BLK, D), jnp.float32),
            pltpu.VMEM((BLK, D), jnp.float32),
            pltpu.VMEM((BLK, 32), jnp.float32),
            pltpu.VMEM((BLK, 32), jnp.float32),
            pltpu.VMEM((BLK,), jnp.int32),
            pltpu.VMEM((BLK,), jnp.int32),
            pltpu.VMEM((BLK, D), jnp.float32),
            pltpu.VMEM((BLK, D), jnp.float32),
            pltpu.VMEM((BLK, 32), jnp.float32),
            pltpu.VMEM((BLK, 32), jnp.float32),
            pltpu.VMEM((BLK, 32), jnp.float32),
            pltpu.SemaphoreType.DMA,
            pltpu.SemaphoreType.DMA,
        ],
    )
    return f(qp, kp, vw, mx, src, dst)


def _combine_kernel(p_ref, bf_ref, o_ref):
    x = p_ref[0] + p_ref[1]
    denom = x[:, 0:8]
    safe = jnp.where(denom != 0.0, denom, 1.0)
    safe3 = jnp.concatenate([safe, safe, safe], axis=1)
    y = x[:, 8:32] / safe3
    r = lax.broadcasted_iota(jnp.int32, (24, 3), 0) // 8
    col = lax.broadcasted_iota(jnp.int32, (24, 3), 1)
    ssum = jnp.where(r == col, 1.0, 0.0)
    o_ref[...] = jnp.dot(y, ssum, preferred_element_type=jnp.float32) + bf_ref[...]


def _combine(partial, bfv):
    nblk = 10
    rb = N_NODE // nblk
    return pl.pallas_call(
        _combine_kernel,
        grid=(nblk,),
        in_specs=[
            pl.BlockSpec((2, rb, 32), lambda i: (0, i, 0)),
            pl.BlockSpec((1, 3), lambda i: (0, 0)),
        ],
        out_specs=pl.BlockSpec((rb, 3), lambda i: (i, 0)),
        out_shape=jax.ShapeDtypeStruct((N_NODE, 3), jnp.float32),
    )(partial, bfv)


def kernel(query, edge_index, edge_diff, pair, ln_q_g, ln_q_b, ln_p_g, ln_p_b,
           Wq, Wk, Wv, Wlb, blb, Wf1, bf1, Wf2, bf2, Wf3, bf3):
    scaling = DH ** -0.5

    wqt_s = Wq.T * scaling
    wkt = Wk.T
    wvt = Wv.T
    dd = jnp.arange(D)
    hh = jnp.arange(H)
    head_mask = jnp.where(dd[:, None] // DH == hh[None, :], 1.0, 0.0)
    m_mat = jnp.concatenate(
        [jnp.zeros((D, 8), jnp.float32),
         Wf1[0][:, None] * head_mask,
         Wf2[0][:, None] * head_mask,
         Wf3[0][:, None] * head_mask], axis=1)

    rr64 = jnp.arange(64)
    gm = jnp.where(rr64[:, None] // 16 == rr64[None, :] // 16, 1.0 / 16, 0.0)
    cc32 = jnp.arange(32)
    w4 = (jnp.tile(Wlb.T, (4, 4))
          * jnp.where(rr64[:, None] // 16 == cc32[None, :] // 8, 1.0, 0.0))
    blb4 = jnp.tile(blb, 4).reshape(1, 32)
    g4 = jnp.tile(ln_p_g, 4).reshape(1, 64)
    b4 = jnp.tile(ln_p_b, 4).reshape(1, 64)
    rr12 = jnp.arange(12)
    cc128 = jnp.arange(128)
    a_mat = jnp.where((cc128[None, :] // 32 == rr12[:, None] // 3)
                      & ((cc128[None, :] % 32) // 8 == rr12[:, None] % 3 + 1),
                      1.0, 0.0)
    rr32 = jnp.arange(32)
    b_mat = jnp.where((cc128[None, :] // 32 == rr32[:, None] // 8)
                      & (cc128[None, :] % 8 == rr32[:, None] % 8),
                      1.0, 0.0)

    n_edge = edge_index.shape[0]
    npad = E_PAD - n_edge
    nblk2 = 10
    rows_in = n_edge // (4 * nblk2)
    rows_out = (E_PAD // 4) // nblk2
    pad_rows = rows_out - rows_in
    pad_ids = (jnp.arange(4 * nblk2 * pad_rows, dtype=jnp.int32)
               % N_NODE).reshape(4, nblk2, pad_rows)
    ei = edge_index.astype(jnp.int32)

    def permute(col):
        full = jnp.concatenate(
            [col.reshape(4, nblk2, rows_in), pad_ids], axis=2)
        return full.reshape(E_PAD)

    src = permute(ei[:, 0])
    dst = permute(ei[:, 1])
    bfv = jnp.concatenate([bf1, bf2, bf3]).reshape(1, 3)

    qp, kp, vw = _node_precompute(query, ln_q_g.reshape(1, D),
                                  ln_q_b.reshape(1, D), wqt_s, wkt, wvt, m_mat)
    mx = _edge_precompute(pair, edge_diff, g4, b4, gm, w4, blb4,
                          a_mat, b_mat)
    partial = _sc_edge_phase(qp, kp, vw, mx, src, dst)
    return _combine(partial, bfv)
